# Initial kernel scaffold; baseline (speedup 1.0000x reference)
import jax
import jax.numpy as jnp
from jax.experimental import pallas as pl


def kernel(hidden_states, gate_w, w_fc, w_proj):
    raise NotImplementedError("write your pallas kernel here")


# speedup vs baseline: 3.2627x; 3.2627x over previous
import functools

import jax
import jax.numpy as jnp
from jax import lax
from jax.experimental import pallas as pl
from jax.experimental.pallas import tpu as pltpu
from jax.experimental.pallas import tpu_sc as plsc

_TOPK = 2
_BLK = 256
_FFT = 512
_NC, _NS = 2, 16
_NW = _NC * _NS


def _logits_body(x_ref, gw_ref, out_ref):
    out_ref[...] = lax.dot_general(
        x_ref[...], gw_ref[...], (((1,), (1,)), ((), ())),
        preferred_element_type=jnp.float32)


def _router_logits(x, gate_w):
    T, H = x.shape
    E = gate_w.shape[0]
    Epad = 128
    gwp = jnp.zeros((Epad, H), gate_w.dtype).at[:E].set(gate_w)
    out = pl.pallas_call(
        _logits_body,
        out_shape=jax.ShapeDtypeStruct((T, Epad), jnp.float32),
    )(x, gwp)
    return out[:, :E]


def _sc_gather(table, idx, n_chunks):
    R = idx.shape[0]
    H = table.shape[1]
    per_w = R // _NW
    C = per_w // n_chunks
    mesh = plsc.VectorSubcoreMesh(
        core_axis_name="c", subcore_axis_name="s",
        num_cores=_NC, num_subcores=_NS)

    @functools.partial(
        pl.kernel,
        out_type=jax.ShapeDtypeStruct((R, H), table.dtype),
        mesh=mesh,
        scratch_types=[
            pltpu.VMEM((C,), jnp.int32),
            pltpu.VMEM((C, H), table.dtype),
            pltpu.SemaphoreType.DMA,
        ],
    )
    def k(table_hbm, idx_hbm, out_hbm, idx_v, rows_v, sem):
        wid = lax.axis_index("s") * _NC + lax.axis_index("c")
        for c in range(n_chunks):
            base = wid * per_w + c * C
            pltpu.sync_copy(idx_hbm.at[pl.ds(base, C)], idx_v)
            pltpu.async_copy(table_hbm.at[idx_v], rows_v, sem).wait()
            pltpu.sync_copy(rows_v, out_hbm.at[pl.ds(base, C)])

    return k(table, idx)


def _ffn_body(be_ref, xs_ref, wfc_ref, wpj_ref, g_ref, out_ref):
    j = pl.program_id(1)
    nf = pl.num_programs(1)
    h = lax.dot_general(
        xs_ref[...], wfc_ref[0], (((1,), (1,)), ((), ())),
        preferred_element_type=jnp.float32)
    h = 0.5 * h * (1.0 + lax.erf(h * 0.7071067811865476))
    contrib = lax.dot_general(
        h, wpj_ref[0], (((1,), (1,)), ((), ())),
        preferred_element_type=jnp.float32)

    @pl.when(j == 0)
    def _():
        out_ref[...] = contrib

    @pl.when(j != 0)
    def _():
        out_ref[...] += contrib

    @pl.when(j == nf - 1)
    def _():
        out_ref[...] *= g_ref[...]


def _grouped_ffn(be, xs, w_fc, w_proj, gates_col):
    PT, H = xs.shape
    E, FF, _ = w_fc.shape
    NB = PT // _BLK
    NF = FF // _FFT
    grid_spec = pltpu.PrefetchScalarGridSpec(
        num_scalar_prefetch=1,
        grid=(NB, NF),
        in_specs=[
            pl.BlockSpec((_BLK, H), lambda i, j, be_r: (i, 0)),
            pl.BlockSpec((1, _FFT, H), lambda i, j, be_r: (be_r[i], j, 0)),
            pl.BlockSpec((1, H, _FFT), lambda i, j, be_r: (be_r[i], 0, j)),
            pl.BlockSpec((_BLK, 1), lambda i, j, be_r: (i, 0)),
        ],
        out_specs=pl.BlockSpec((_BLK, H), lambda i, j, be_r: (i, 0)),
    )
    return pl.pallas_call(
        _ffn_body,
        grid_spec=grid_spec,
        out_shape=jax.ShapeDtypeStruct((PT, H), jnp.float32),
        compiler_params=pltpu.CompilerParams(
            dimension_semantics=("arbitrary", "arbitrary")),
    )(be, xs, w_fc, w_proj, gates_col)


def _pair_body(in_ref, out_ref):
    out_ref[...] = in_ref[:, 0, :] + in_ref[:, 1, :]


def _pair_sum(pairs):
    T, K, H = pairs.shape
    BT = 512
    return pl.pallas_call(
        _pair_body,
        grid=(T // BT,),
        in_specs=[pl.BlockSpec((BT, K, H), lambda i: (i, 0, 0))],
        out_specs=pl.BlockSpec((BT, H), lambda i: (i, 0)),
        out_shape=jax.ShapeDtypeStruct((T, H), pairs.dtype),
    )(pairs)


def kernel(hidden_states, gate_w, w_fc, w_proj):
    Bq, Sq, H = hidden_states.shape
    E, FF, _ = w_fc.shape
    T = Bq * Sq
    TK = _TOPK
    NS = T * TK

    x = hidden_states.reshape(T, H)
    logits = _router_logits(x, gate_w)

    top_logits, top_idx = lax.top_k(logits, TK)
    gates = jax.nn.softmax(top_logits, axis=1).astype(x.dtype)
    tke = top_idx.reshape(-1).astype(jnp.int32)
    order = jnp.argsort(tke, stable=True).astype(jnp.int32)
    sorted_experts = tke[order]
    bidx = (order // TK).astype(jnp.int32)
    batch_gates = gates.reshape(-1)[order]

    NB = NS // _BLK + E
    PT = NB * _BLK
    g = jnp.bincount(tke, length=E).astype(jnp.int32)
    o = jnp.concatenate([jnp.zeros((1,), jnp.int32), jnp.cumsum(g)[:-1]])
    bpe = (g + _BLK - 1) // _BLK
    po = jnp.concatenate([jnp.zeros((1,), jnp.int32),
                          jnp.cumsum(bpe)[:-1]]) * _BLK
    p = jnp.arange(NS, dtype=jnp.int32)
    ppos = p - o[sorted_experts] + po[sorted_experts]
    bidx_pad = jnp.zeros((PT,), jnp.int32).at[ppos].set(bidx)
    gates_pad = jnp.zeros((PT,), jnp.float32).at[ppos].set(batch_gates)
    be = jnp.repeat(jnp.arange(E, dtype=jnp.int32), bpe,
                    total_repeat_length=NB)

    xs = _sc_gather(x, bidx_pad, 2)
    outs = _grouped_ffn(be, xs, w_fc, w_proj, gates_pad[:, None])

    inv = jnp.zeros((NS,), jnp.int32).at[order].set(p)
    pos_pairs = ppos[inv]
    gathered = _sc_gather(outs, pos_pairs, 2)
    result = _pair_sum(gathered.reshape(T, TK, H))

    return (result.reshape(Bq, Sq, H), logits)
